# baseline (speedup 1.0000x reference)
import functools

import jax
import jax.numpy as jnp
from jax import lax
from jax.experimental import pallas as pl
from jax.experimental.pallas import tpu as pltpu
from jax.experimental.pallas import tpu_sc as plsc

_B = 16384
_NCAT = 26
_CARD = 1000
_EMB = 500
_BIO = 768
_NNUM = 13
_F1 = 1024
_F2 = 512
_EPS = 1e-5
_EMB_ROWS = _NCAT * _CARD
_NART = 10000
_TBL_ROWS = _EMB_ROWS + _NART
_NIDX = _NCAT + 1
_IDXPAD = 32

_NC = 2
_NS = 16
_NW = _NC * _NS
_BPW = _B // _NW
_OCH = 8
_LANES = _F1 // 16


def _proj_body(emb_ref, w1e_ref, bio_ref, w1b_ref, out_ref):
    i = pl.program_id(0)

    @pl.when(i < _NCAT)
    def _():
        out_ref[...] = jnp.dot(emb_ref[0].astype(jnp.bfloat16),
                               w1e_ref[0].astype(jnp.bfloat16),
                               preferred_element_type=jnp.float32)

    @pl.when(i >= _NCAT)
    def _():
        out_ref[...] = jnp.dot(bio_ref[...].astype(jnp.bfloat16),
                               w1b_ref[...].astype(jnp.bfloat16),
                               preferred_element_type=jnp.float32)


def _build_table(emb_tables, w1e, bio, w1b):
    nbio = _NART // _CARD
    return pl.pallas_call(
        _proj_body,
        grid=(_NCAT + nbio,),
        in_specs=[
            pl.BlockSpec((1, _CARD, _EMB),
                         lambda i: (jnp.minimum(i, _NCAT - 1), 0, 0)),
            pl.BlockSpec((1, _EMB, _F1),
                         lambda i: (jnp.minimum(i, _NCAT - 1), 0, 0)),
            pl.BlockSpec((_CARD, _BIO),
                         lambda i: (jnp.maximum(i - _NCAT, 0), 0)),
            pl.BlockSpec((_BIO, _F1), lambda i: (0, 0)),
        ],
        out_specs=pl.BlockSpec((_CARD, _F1), lambda i: (i, 0)),
        out_shape=jax.ShapeDtypeStruct((_TBL_ROWS, _F1), jnp.float32),
    )(emb_tables, w1e, bio, w1b)


_CHS = 128
_NCH = _BPW // _CHS
_NACC = 32
_NPASS = _F1 // (16 * _NACC)
_HW = _F1 // 2
_GW = _NIDX * 2 + 2


def _sc_gather_sum(tbl, idx_all):
    mesh = plsc.VectorSubcoreMesh(core_axis_name="c", subcore_axis_name="s")

    @functools.partial(
        pl.kernel, mesh=mesh,
        out_type=jax.ShapeDtypeStruct((_B, _F1), jnp.float32),
        scratch_types=[
            pltpu.VMEM((2, _CHS, _GW), jnp.int32),
            pltpu.VMEM((2, _GW, _HW), jnp.float32),
            pltpu.VMEM((_OCH, _F1), jnp.float32),
            pltpu.SemaphoreType.DMA,
            pltpu.SemaphoreType.DMA,
        ],
    )
    def k(tbl_hbm, idx_hbm, out_hbm, idx_c, rows_v, outb_v, sem0, sem1):
        wid = lax.axis_index("s") * _NC + lax.axis_index("c")
        base = pl.multiple_of(wid * _BPW, _BPW)
        sems = (sem0, sem1)

        def start(b, cp, srow):
            pltpu.async_copy(tbl_hbm.at[idx_c.at[cp, srow]],
                             rows_v.at[b], sems[b])

        def process(s, b):
            pltpu.make_async_copy(tbl_hbm.at[idx_c.at[0, 0]],
                                  rows_v.at[b], sems[b]).wait()
            o = s & (_OCH - 1)

            def pbody(p, carry, b=b, o=o):
                def rbody(t, accs, b=b, p=p):
                    r = 1 + t * 2
                    for rr in (r, r + 1):
                        accs = [accs[j]
                                + rows_v[b, 2 * rr + p, pl.ds(j * 16, 16)]
                                for j in range(_NACC)]
                    return accs

                accs = [rows_v[b, p, pl.ds(j * 16, 16)]
                        for j in range(_NACC)]
                accs = lax.fori_loop(0, (_NIDX - 1) // 2, rbody, accs)
                for j in range(_NACC):
                    outb_v[o, pl.ds(p * 512 + j * 16, 16)] = accs[j]
                return carry

            lax.fori_loop(0, _NPASS, pbody, 0)

            @pl.when(o == _OCH - 1)
            def _():
                off = pl.multiple_of(base + s - (_OCH - 1), _OCH)
                pltpu.sync_copy(outb_v, out_hbm.at[pl.ds(off, _OCH)])

        pltpu.sync_copy(idx_hbm.at[pl.ds(base, _CHS)], idx_c.at[0])
        start(0, 0, 0)
        start(1, 0, 1)

        for ch in range(_NCH):
            cp = ch & 1
            if ch + 1 < _NCH:
                pltpu.sync_copy(
                    idx_hbm.at[pl.ds(base + (ch + 1) * _CHS, _CHS)],
                    idx_c.at[(ch + 1) & 1])

            def pairb(j, carry, ch=ch, cp=cp):
                for b in range(2):
                    sl = j * 2 + b
                    process(ch * _CHS + sl, b)
                    start(b, cp, sl + 2)
                return carry
            lax.fori_loop(0, _CHS // 2 - 1, pairb, 0)

            for b in range(2):
                sl = _CHS - 2 + b
                process(ch * _CHS + sl, b)
                if ch + 1 < _NCH:
                    start(b, (ch + 1) & 1, b)

    return k(tbl, idx_all)


_BB = 512


def _stats_body(h_ref, xn_ref, wn_ref, b1_ref, out_ref):
    i = pl.program_id(0)
    h = (h_ref[...]
         + jnp.dot(xn_ref[...], wn_ref[...], preferred_element_type=jnp.float32)
         + b1_ref[...][None, :])
    s1 = jnp.sum(h, axis=0)
    s2 = jnp.sum(h * h, axis=0)

    @pl.when(i == 0)
    def _():
        out_ref[...] = jnp.zeros_like(out_ref)

    out_ref[...] += jnp.concatenate([s1[None], s2[None]], axis=0)


def _stats_call(h1p, x_num, w1n, b1):
    return pl.pallas_call(
        _stats_body,
        grid=(_B // _BB,),
        in_specs=[
            pl.BlockSpec((_BB, _F1), lambda i: (i, 0)),
            pl.BlockSpec((_BB, _NNUM), lambda i: (i, 0)),
            pl.BlockSpec((_NNUM, _F1), lambda i: (0, 0)),
            pl.BlockSpec((_F1,), lambda i: (0,)),
        ],
        out_specs=pl.BlockSpec((2, _F1), lambda i: (0, 0)),
        out_shape=jax.ShapeDtypeStruct((2, _F1), jnp.float32),
    )(h1p, x_num, w1n, b1)


def _mid_body(h_ref, xn_ref, wn_ref, b1_ref, st_ref, g1_ref, be1_ref,
              w2_ref, b2_ref, h2_ref, st2_ref):
    i = pl.program_id(0)
    h = (h_ref[...]
         + jnp.dot(xn_ref[...], wn_ref[...], preferred_element_type=jnp.float32)
         + b1_ref[...][None, :])
    mu = st_ref[0] * (1.0 / _B)
    var = st_ref[1] * (1.0 / _B) - mu * mu
    sc = g1_ref[...] * lax.rsqrt(var + _EPS)
    sh = be1_ref[...] - mu * sc
    a = jnp.maximum(h * sc[None, :] + sh[None, :], 0.0)
    h2 = (jnp.dot(a, w2_ref[...], preferred_element_type=jnp.float32)
          + b2_ref[...][None, :])
    h2_ref[...] = h2

    @pl.when(i == 0)
    def _():
        st2_ref[...] = jnp.zeros_like(st2_ref)

    st2_ref[...] += jnp.concatenate(
        [jnp.sum(h2, axis=0)[None], jnp.sum(h2 * h2, axis=0)[None]], axis=0)


def _mid_call(h1p, x_num, w1n, b1, st1, g1, be1, w2, b2):
    return pl.pallas_call(
        _mid_body,
        grid=(_B // _BB,),
        in_specs=[
            pl.BlockSpec((_BB, _F1), lambda i: (i, 0)),
            pl.BlockSpec((_BB, _NNUM), lambda i: (i, 0)),
            pl.BlockSpec((_NNUM, _F1), lambda i: (0, 0)),
            pl.BlockSpec((_F1,), lambda i: (0,)),
            pl.BlockSpec((2, _F1), lambda i: (0, 0)),
            pl.BlockSpec((_F1,), lambda i: (0,)),
            pl.BlockSpec((_F1,), lambda i: (0,)),
            pl.BlockSpec((_F1, _F2), lambda i: (0, 0)),
            pl.BlockSpec((_F2,), lambda i: (0,)),
        ],
        out_specs=[
            pl.BlockSpec((_BB, _F2), lambda i: (i, 0)),
            pl.BlockSpec((2, _F2), lambda i: (0, 0)),
        ],
        out_shape=[
            jax.ShapeDtypeStruct((_B, _F2), jnp.float32),
            jax.ShapeDtypeStruct((2, _F2), jnp.float32),
        ],
    )(h1p, x_num, w1n, b1, st1, g1, be1, w2, b2)


def _tail_body(h2_ref, st2_ref, g2_ref, be2_ref, w3_ref, b3_ref, xn_ref,
               out_ref):
    mu = st2_ref[0] * (1.0 / _B)
    var = st2_ref[1] * (1.0 / _B) - mu * mu
    sc = g2_ref[...] * lax.rsqrt(var + _EPS)
    sh = be2_ref[...] - mu * sc
    a = jnp.maximum(h2_ref[...] * sc[None, :] + sh[None, :], 0.0)
    out_ref[...] = (jnp.dot(a, w3_ref[...], preferred_element_type=jnp.float32)
                    + b3_ref[...]
                    + xn_ref[..., _NNUM - 1:_NNUM])


def _tail_call(h2, st2, g2, be2, w3, b3, x_num):
    return pl.pallas_call(
        _tail_body,
        grid=(_B // _BB,),
        in_specs=[
            pl.BlockSpec((_BB, _F2), lambda i: (i, 0)),
            pl.BlockSpec((2, _F2), lambda i: (0, 0)),
            pl.BlockSpec((_F2,), lambda i: (0,)),
            pl.BlockSpec((_F2,), lambda i: (0,)),
            pl.BlockSpec((_F2, 1), lambda i: (0, 0)),
            pl.BlockSpec((1, 1), lambda i: (0, 0)),
            pl.BlockSpec((_BB, _NNUM), lambda i: (i, 0)),
        ],
        out_specs=pl.BlockSpec((_BB, 1), lambda i: (i, 0)),
        out_shape=jax.ShapeDtypeStruct((_B, 1), jnp.float32),
    )(h2, st2, g2, be2, w3, b3, x_num)


def kernel(x_cat, x_num, artist_ids, biography_encoded, emb_tables,
           W1, b1, g1, be1, W2, b2, g2, be2, W3, b3):
    n_e = _NCAT * _EMB
    w1e = W1[:n_e].reshape(_NCAT, _EMB, _F1)
    w1b = W1[n_e:n_e + _BIO]
    w1n = W1[n_e + _BIO:]

    tbl = _build_table(emb_tables, w1e, biography_encoded, w1b)
    tbl2 = tbl.reshape(_TBL_ROWS * 2, _HW)

    offs = jnp.arange(_NCAT, dtype=jnp.int32) * _CARD
    idx27 = jnp.concatenate(
        [x_cat.astype(jnp.int32) + offs[None, :],
         artist_ids.astype(jnp.int32)[:, None] + _EMB_ROWS], axis=1)
    idx54 = (idx27[:, :, None] * 2
             + jnp.arange(2, dtype=jnp.int32)[None, None, :]
             ).reshape(_B, _NIDX * 2)
    pad = ((jnp.arange(_B, dtype=jnp.int32)[:, None] * 2
            + jnp.arange(2, dtype=jnp.int32)[None, :]) % (_TBL_ROWS * 2))
    idx_all = jnp.concatenate([idx54, pad], axis=1)

    h1p = _sc_gather_sum(tbl2, idx_all)

    st1 = _stats_call(h1p, x_num, w1n, b1)
    h2, st2 = _mid_call(h1p, x_num, w1n, b1, st1, g1, be1, W2, b2)
    out = _tail_call(h2, st2, g2, be2, W3, b3.reshape(1, 1), x_num)
    return out


# speedup vs baseline: 1.0996x; 1.0996x over previous
import functools

import jax
import jax.numpy as jnp
from jax import lax
from jax.experimental import pallas as pl
from jax.experimental.pallas import tpu as pltpu
from jax.experimental.pallas import tpu_sc as plsc

_B = 16384
_NCAT = 26
_CARD = 1000
_EMB = 500
_BIO = 768
_NNUM = 13
_F1 = 1024
_F2 = 512
_EPS = 1e-5
_EMB_ROWS = _NCAT * _CARD
_NART = 10000
_TBL_ROWS = _EMB_ROWS + _NART
_NIDX = _NCAT + 1
_IDXPAD = 32

_NC = 2
_NS = 16
_NW = _NC * _NS
_BPW = _B // _NW
_OCH = 8
_LANES = _F1 // 16


def _proj_body(emb_ref, w1e_ref, bio_ref, w1b_ref, out_ref):
    i = pl.program_id(0)

    @pl.when(i < _NCAT)
    def _():
        out_ref[...] = jnp.dot(emb_ref[0].astype(jnp.bfloat16),
                               w1e_ref[0].astype(jnp.bfloat16),
                               preferred_element_type=jnp.float32)

    @pl.when(i >= _NCAT)
    def _():
        out_ref[...] = jnp.dot(bio_ref[...].astype(jnp.bfloat16),
                               w1b_ref[...].astype(jnp.bfloat16),
                               preferred_element_type=jnp.float32)


def _build_table(emb_tables, w1e, bio, w1b):
    nbio = _NART // _CARD
    return pl.pallas_call(
        _proj_body,
        grid=(_NCAT + nbio,),
        in_specs=[
            pl.BlockSpec((1, _CARD, _EMB),
                         lambda i: (jnp.minimum(i, _NCAT - 1), 0, 0)),
            pl.BlockSpec((1, _EMB, _F1),
                         lambda i: (jnp.minimum(i, _NCAT - 1), 0, 0)),
            pl.BlockSpec((_CARD, _BIO),
                         lambda i: (jnp.maximum(i - _NCAT, 0), 0)),
            pl.BlockSpec((_BIO, _F1), lambda i: (0, 0)),
        ],
        out_specs=pl.BlockSpec((_CARD, _F1), lambda i: (i, 0)),
        out_shape=jax.ShapeDtypeStruct((_TBL_ROWS, _F1), jnp.float32),
    )(emb_tables, w1e, bio, w1b)


_CHS = 128
_NCH = _BPW // _CHS
_NACC = 32
_NPASS = _F1 // (16 * _NACC)


def _sc_gather_sum(tbl, idx_all):
    mesh = plsc.VectorSubcoreMesh(core_axis_name="c", subcore_axis_name="s")

    @functools.partial(
        pl.kernel, mesh=mesh,
        out_type=jax.ShapeDtypeStruct((_B, _F1), jnp.float32),
        scratch_types=[
            pltpu.VMEM((2, _CHS, _IDXPAD), jnp.int32),
            pltpu.VMEM((2, _IDXPAD, _F1), jnp.float32),
            pltpu.VMEM((2, _OCH // 2, _F1), jnp.float32),
            pltpu.SemaphoreType.DMA,
            pltpu.SemaphoreType.DMA,
            pltpu.SemaphoreType.DMA,
            pltpu.SemaphoreType.DMA,
        ],
    )
    def k(tbl_hbm, idx_hbm, out_hbm, idx_c, rows_v, outb_v,
          sem0, sem1, osem0, osem1):
        wid = lax.axis_index("s") * _NC + lax.axis_index("c")
        base = pl.multiple_of(wid * _BPW, _BPW)
        sems = (sem0, sem1)

        def start(b, cp, srow):
            pltpu.async_copy(tbl_hbm.at[idx_c.at[cp, srow]],
                             rows_v.at[b], sems[b])

        osems = (osem0, osem1)
        _HF = _OCH // 2

        def process(s, b):
            pltpu.make_async_copy(tbl_hbm.at[idx_c.at[0, 0]],
                                  rows_v.at[b], sems[b]).wait()
            o = s & (_HF - 1)
            half = (s // _HF) & 1

            for hf in range(2):
                @pl.when((o == 0) & (half == hf) & (s >= 2 * _HF))
                def _(hf=hf):
                    pltpu.make_async_copy(
                        outb_v.at[hf], out_hbm.at[pl.ds(0, _HF)],
                        osems[hf]).wait()

            def pbody(p, carry, b=b, o=o, half=half):
                def rbody(t, accs, b=b, p=p):
                    r = 1 + t * 2
                    for rr in (r, r + 1):
                        accs = [accs[j]
                                + rows_v[b, rr, pl.ds(p * 512 + j * 16, 16)]
                                for j in range(_NACC)]
                    return accs

                accs = [rows_v[b, 0, pl.ds(p * 512 + j * 16, 16)]
                        for j in range(_NACC)]
                accs = lax.fori_loop(0, (_NIDX - 1) // 2, rbody, accs)
                for j in range(_NACC):
                    outb_v[half, o, pl.ds(p * 512 + j * 16, 16)] = accs[j]
                return carry

            lax.fori_loop(0, _NPASS, pbody, 0)

            for hf in range(2):
                @pl.when((o == _HF - 1) & (half == hf))
                def _(s=s, hf=hf):
                    off = pl.multiple_of(base + s - (_HF - 1), _HF)
                    pltpu.async_copy(outb_v.at[hf],
                                     out_hbm.at[pl.ds(off, _HF)], osems[hf])


        pltpu.sync_copy(idx_hbm.at[pl.ds(base, _CHS)], idx_c.at[0])
        start(0, 0, 0)
        start(1, 0, 1)

        for ch in range(_NCH):
            cp = ch & 1
            if ch + 1 < _NCH:
                pltpu.sync_copy(
                    idx_hbm.at[pl.ds(base + (ch + 1) * _CHS, _CHS)],
                    idx_c.at[(ch + 1) & 1])

            def pairb(j, carry, ch=ch, cp=cp):
                for b in range(2):
                    sl = j * 2 + b
                    process(ch * _CHS + sl, b)
                    start(b, cp, sl + 2)
                return carry
            lax.fori_loop(0, _CHS // 2 - 1, pairb, 0)

            for b in range(2):
                sl = _CHS - 2 + b
                process(ch * _CHS + sl, b)
                if ch + 1 < _NCH:
                    start(b, (ch + 1) & 1, b)

        pltpu.make_async_copy(outb_v.at[0], out_hbm.at[pl.ds(0, _OCH // 2)],
                              osem0).wait()
        pltpu.make_async_copy(outb_v.at[1], out_hbm.at[pl.ds(0, _OCH // 2)],
                              osem1).wait()

    return k(tbl, idx_all)


_BB = 512


def _stats_body(h_ref, xn_ref, wn_ref, b1_ref, out_ref):
    i = pl.program_id(0)
    h = (h_ref[...]
         + jnp.dot(xn_ref[...], wn_ref[...], preferred_element_type=jnp.float32)
         + b1_ref[...][None, :])
    s1 = jnp.sum(h, axis=0)
    s2 = jnp.sum(h * h, axis=0)

    @pl.when(i == 0)
    def _():
        out_ref[...] = jnp.zeros_like(out_ref)

    out_ref[...] += jnp.concatenate([s1[None], s2[None]], axis=0)


def _stats_call(h1p, x_num, w1n, b1):
    return pl.pallas_call(
        _stats_body,
        grid=(_B // _BB,),
        in_specs=[
            pl.BlockSpec((_BB, _F1), lambda i: (i, 0)),
            pl.BlockSpec((_BB, _NNUM), lambda i: (i, 0)),
            pl.BlockSpec((_NNUM, _F1), lambda i: (0, 0)),
            pl.BlockSpec((_F1,), lambda i: (0,)),
        ],
        out_specs=pl.BlockSpec((2, _F1), lambda i: (0, 0)),
        out_shape=jax.ShapeDtypeStruct((2, _F1), jnp.float32),
    )(h1p, x_num, w1n, b1)


def _mid_body(h_ref, xn_ref, wn_ref, b1_ref, st_ref, g1_ref, be1_ref,
              w2_ref, b2_ref, h2_ref, st2_ref):
    i = pl.program_id(0)
    h = (h_ref[...]
         + jnp.dot(xn_ref[...], wn_ref[...], preferred_element_type=jnp.float32)
         + b1_ref[...][None, :])
    mu = st_ref[0] * (1.0 / _B)
    var = st_ref[1] * (1.0 / _B) - mu * mu
    sc = g1_ref[...] * lax.rsqrt(var + _EPS)
    sh = be1_ref[...] - mu * sc
    a = jnp.maximum(h * sc[None, :] + sh[None, :], 0.0)
    h2 = (jnp.dot(a, w2_ref[...], preferred_element_type=jnp.float32)
          + b2_ref[...][None, :])
    h2_ref[...] = h2

    @pl.when(i == 0)
    def _():
        st2_ref[...] = jnp.zeros_like(st2_ref)

    st2_ref[...] += jnp.concatenate(
        [jnp.sum(h2, axis=0)[None], jnp.sum(h2 * h2, axis=0)[None]], axis=0)


def _mid_call(h1p, x_num, w1n, b1, st1, g1, be1, w2, b2):
    return pl.pallas_call(
        _mid_body,
        grid=(_B // _BB,),
        in_specs=[
            pl.BlockSpec((_BB, _F1), lambda i: (i, 0)),
            pl.BlockSpec((_BB, _NNUM), lambda i: (i, 0)),
            pl.BlockSpec((_NNUM, _F1), lambda i: (0, 0)),
            pl.BlockSpec((_F1,), lambda i: (0,)),
            pl.BlockSpec((2, _F1), lambda i: (0, 0)),
            pl.BlockSpec((_F1,), lambda i: (0,)),
            pl.BlockSpec((_F1,), lambda i: (0,)),
            pl.BlockSpec((_F1, _F2), lambda i: (0, 0)),
            pl.BlockSpec((_F2,), lambda i: (0,)),
        ],
        out_specs=[
            pl.BlockSpec((_BB, _F2), lambda i: (i, 0)),
            pl.BlockSpec((2, _F2), lambda i: (0, 0)),
        ],
        out_shape=[
            jax.ShapeDtypeStruct((_B, _F2), jnp.float32),
            jax.ShapeDtypeStruct((2, _F2), jnp.float32),
        ],
    )(h1p, x_num, w1n, b1, st1, g1, be1, w2, b2)


def _tail_body(h2_ref, st2_ref, g2_ref, be2_ref, w3_ref, b3_ref, xn_ref,
               out_ref):
    mu = st2_ref[0] * (1.0 / _B)
    var = st2_ref[1] * (1.0 / _B) - mu * mu
    sc = g2_ref[...] * lax.rsqrt(var + _EPS)
    sh = be2_ref[...] - mu * sc
    a = jnp.maximum(h2_ref[...] * sc[None, :] + sh[None, :], 0.0)
    out_ref[...] = (jnp.dot(a, w3_ref[...], preferred_element_type=jnp.float32)
                    + b3_ref[...]
                    + xn_ref[..., _NNUM - 1:_NNUM])


def _tail_call(h2, st2, g2, be2, w3, b3, x_num):
    return pl.pallas_call(
        _tail_body,
        grid=(_B // _BB,),
        in_specs=[
            pl.BlockSpec((_BB, _F2), lambda i: (i, 0)),
            pl.BlockSpec((2, _F2), lambda i: (0, 0)),
            pl.BlockSpec((_F2,), lambda i: (0,)),
            pl.BlockSpec((_F2,), lambda i: (0,)),
            pl.BlockSpec((_F2, 1), lambda i: (0, 0)),
            pl.BlockSpec((1, 1), lambda i: (0, 0)),
            pl.BlockSpec((_BB, _NNUM), lambda i: (i, 0)),
        ],
        out_specs=pl.BlockSpec((_BB, 1), lambda i: (i, 0)),
        out_shape=jax.ShapeDtypeStruct((_B, 1), jnp.float32),
    )(h2, st2, g2, be2, w3, b3, x_num)


def kernel(x_cat, x_num, artist_ids, biography_encoded, emb_tables,
           W1, b1, g1, be1, W2, b2, g2, be2, W3, b3):
    n_e = _NCAT * _EMB
    w1e = W1[:n_e].reshape(_NCAT, _EMB, _F1)
    w1b = W1[n_e:n_e + _BIO]
    w1n = W1[n_e + _BIO:]

    tbl = _build_table(emb_tables, w1e, biography_encoded, w1b)

    offs = jnp.arange(_NCAT, dtype=jnp.int32) * _CARD
    idx27 = jnp.concatenate(
        [x_cat.astype(jnp.int32) + offs[None, :],
         artist_ids.astype(jnp.int32)[:, None] + _EMB_ROWS], axis=1)
    npad = _IDXPAD - _NIDX
    pad = ((jnp.arange(_B, dtype=jnp.int32)[:, None] * npad
            + jnp.arange(npad, dtype=jnp.int32)[None, :]) % _TBL_ROWS)
    idx_all = jnp.concatenate([idx27, pad], axis=1)

    h1p = _sc_gather_sum(tbl, idx_all)

    st1 = _stats_call(h1p, x_num, w1n, b1)
    h2, st2 = _mid_call(h1p, x_num, w1n, b1, st1, g1, be1, W2, b2)
    out = _tail_call(h2, st2, g2, be2, W3, b3.reshape(1, 1), x_num)
    return out
